# baseline (speedup 1.0000x reference)
import functools

import jax
import jax.numpy as jnp
from jax import lax
from jax.experimental import pallas as pl
from jax.experimental.pallas import tpu as pltpu
from jax.experimental.pallas import tpu_sc as plsc

_N = 10000
_D = 128
_E = 320000
_NT = 32
_CHUNK = 128
_NCH = 78
_EPT = _NCH * _CHUNK
_EXTRA_OFF = _NT * _EPT
_NEXTRA = (_E - _EXTRA_OFF) // _CHUNK
_ROWS_PT = 632
_NPAD = _N


def _matmul_body(x_ref, w_ref, o_ref):
    o_ref[...] = jnp.dot(x_ref[...], w_ref[...],
                         preferred_element_type=jnp.float32)


def _support_matmul(x, W):
    bm = 2000
    return pl.pallas_call(
        _matmul_body,
        out_shape=jax.ShapeDtypeStruct((_N, _D), jnp.float32),
        grid=(_N // bm,),
        in_specs=[pl.BlockSpec((bm, _D), lambda i: (i, 0)),
                  pl.BlockSpec((_D, _D), lambda i: (0, 0))],
        out_specs=pl.BlockSpec((bm, _D), lambda i: (i, 0)),
    )(x, W)


def _combine_body(p_ref, b_ref, o_ref):
    o_ref[...] = p_ref[0] + p_ref[1] + b_ref[...]


def _combine(partials, b2d):
    bm = 2000
    return pl.pallas_call(
        _combine_body,
        out_shape=jax.ShapeDtypeStruct((_N, _D), jnp.float32),
        grid=(_N // bm,),
        in_specs=[pl.BlockSpec((2, bm, _D), lambda i: (0, i, 0)),
                  pl.BlockSpec((1, _D), lambda i: (0, 0))],
        out_specs=pl.BlockSpec((bm, _D), lambda i: (i, 0)),
    )(partials, b2d)


def _sc_aggregate(support, ei, w):
    mesh = plsc.VectorSubcoreMesh(core_axis_name="c", subcore_axis_name="s")

    @functools.partial(
        pl.kernel,
        out_type=jax.ShapeDtypeStruct((2, _NPAD, _D), jnp.float32),
        mesh=mesh,
        scratch_types=[
            pltpu.VMEM((2, _CHUNK), jnp.int32),
            pltpu.VMEM((2, _CHUNK), jnp.int32),
            pltpu.VMEM((2, _CHUNK), jnp.int32),
            pltpu.VMEM((_CHUNK,), jnp.float32),
            pltpu.VMEM((_CHUNK,), jnp.float32),
            pltpu.VMEM((_CHUNK,), jnp.float32),
            pltpu.VMEM((_CHUNK, _D), jnp.float32),
            pltpu.VMEM((_CHUNK, _D), jnp.float32),
            pltpu.VMEM((_CHUNK, _D), jnp.float32),
            pltpu.VMEM_SHARED((_NPAD, _D), jnp.float32),
            pltpu.SemaphoreType.DMA,
            pltpu.SemaphoreType.DMA,
            pltpu.SemaphoreType.DMA,
            pltpu.SemaphoreType.DMA,
            pltpu.SemaphoreType.DMA,
            pltpu.SemaphoreType.DMA,
            pltpu.SemaphoreType.DMA,
            pltpu.SemaphoreType.DMA,
            pltpu.SemaphoreType.DMA,
        ],
    )
    def k(sup_hbm, ei_hbm, w_hbm, out_hbm,
          ebuf0, ebuf1, ebuf2, wbuf0, wbuf1, wbuf2, buf0, buf1, buf2, acc,
          esem0, esem1, esem2, gsem0, gsem1, gsem2, ssem0, ssem1, ssem2):
        cid = lax.axis_index("c")
        sid = lax.axis_index("s")
        wid = cid * 16 + sid
        base = wid * _EPT
        ebufs = (ebuf0, ebuf1, ebuf2)
        wbufs = (wbuf0, wbuf1, wbuf2)
        bufs = (buf0, buf1, buf2)
        esems = (esem0, esem1, esem2)
        gsems = (gsem0, gsem1, gsem2)
        ssems = (ssem0, ssem1, ssem2)

        def edge_start_at(off, q):
            pltpu.make_async_copy(ei_hbm.at[:, pl.ds(off, _CHUNK)],
                                  ebufs[q], esems[q]).start()
            pltpu.make_async_copy(w_hbm.at[pl.ds(off, _CHUNK)],
                                  wbufs[q], esems[q]).start()

        def edge_start(ci, q):
            edge_start_at(base + ci * _CHUNK, q)

        def edge_wait(q):
            pltpu.make_async_copy(ei_hbm.at[:, pl.ds(0, _CHUNK)],
                                  ebufs[q], esems[q]).wait()
            pltpu.make_async_copy(w_hbm.at[pl.ds(0, _CHUNK)],
                                  wbufs[q], esems[q]).wait()

        def gather_start(q):
            pltpu.make_async_copy(sup_hbm.at[ebufs[q].at[1]],
                                  bufs[q], gsems[q]).start()

        def gather_wait(q):
            pltpu.make_async_copy(sup_hbm.at[ebufs[q].at[1]],
                                  bufs[q], gsems[q]).wait()

        def scatter_start(q):
            pltpu.async_copy(bufs[q], acc.at[ebufs[q].at[0]], ssems[q],
                             add=True)

        def scatter_wait(q):
            pltpu.make_async_copy(sup_hbm.at[pl.ds(0, _CHUNK)], bufs[q],
                                  ssems[q]).wait()

        edge_start(0, 0)
        edge_start(1, 1)
        edge_wait(0)
        gather_start(0)
        edge_wait(1)
        gather_start(1)

        zeros16 = jnp.zeros((16,), jnp.float32)

        def zrow(r, carry):
            for j in range(8):
                buf2[r, pl.ds(j * 16, 16)] = zeros16
            return carry

        lax.fori_loop(0, _CHUNK, zrow, 0)
        row0 = jnp.minimum(sid * _ROWS_PT, _NPAD - _ROWS_PT)
        nz = _ROWS_PT // _CHUNK
        rem = _ROWS_PT - nz * _CHUNK
        for q in range(nz):
            pltpu.make_async_copy(
                buf2, acc.at[pl.ds(row0 + q * _CHUNK, _CHUNK)], gsem2).start()
        pltpu.make_async_copy(
            buf2.at[pl.ds(0, rem)],
            acc.at[pl.ds(row0 + nz * _CHUNK, rem)], gsem2).start()
        for q in range(nz):
            pltpu.make_async_copy(
                buf2, acc.at[pl.ds(row0, _CHUNK)], gsem2).wait()
        pltpu.make_async_copy(
            buf2.at[pl.ds(0, rem)], acc.at[pl.ds(row0, rem)], gsem2).wait()
        plsc.subcore_barrier()

        def mul_chunk(q):
            wbuf, buf = wbufs[q], bufs[q]

            @plsc.parallel_loop(0, _CHUNK // 16, unroll=2)
            def gbody(g):
                wv = wbuf[pl.ds(g * 16, 16)]
                for l in range(16):
                    wl = wv[l]
                    e = g * 16 + l
                    for j in range(8):
                        sl = pl.ds(j * 16, 16)
                        buf[e, sl] = buf[e, sl] * wl

        def phase(i, q, r):
            gather_wait(q)
            mul_chunk(q)
            scatter_start(q)

            @pl.when(i >= 1)
            def _ret():
                scatter_wait(r)

            @pl.when(i + 2 < _NCH)
            def _pre():
                edge_start(i + 2, r)
                edge_wait(r)
                gather_start(r)

        def body(kk, carry):
            i0 = kk * 3
            phase(i0, 0, 2)
            phase(i0 + 1, 1, 0)
            phase(i0 + 2, 2, 1)
            return carry

        lax.fori_loop(0, _NCH // 3, body, 0)
        scatter_wait((_NCH - 1) % 3)

        eidx = sid * 2 + cid
        @pl.when(sid < _NEXTRA // 2)
        def _extra():
            edge_start_at(_EXTRA_OFF + eidx * _CHUNK, 0)
            edge_wait(0)
            gather_start(0)
            gather_wait(0)
            mul_chunk(0)
            pltpu.sync_copy(buf0, acc.at[ebuf0.at[0]], add=True)

        plsc.subcore_barrier()
        pltpu.sync_copy(acc.at[pl.ds(row0, _ROWS_PT)],
                        out_hbm.at[cid, pl.ds(row0, _ROWS_PT)])

    return k(support, ei, w)


def kernel(x, edge_index, edge_weight, W, b):
    ei = jnp.asarray(edge_index, jnp.int32)
    w = jnp.asarray(edge_weight, jnp.float32)

    support = _support_matmul(x, W)
    partials = _sc_aggregate(support, ei, w)
    return _combine(partials, b.reshape(1, _D))


# speedup vs baseline: 1.0220x; 1.0220x over previous
import functools

import jax
import jax.numpy as jnp
from jax import lax
from jax.experimental import pallas as pl
from jax.experimental.pallas import tpu as pltpu
from jax.experimental.pallas import tpu_sc as plsc

_N = 10000
_D = 128
_E = 320000
_NT = 32
_CHUNK = 128
_NCH = 78
_EPT = _NCH * _CHUNK
_EXTRA_OFF = _NT * _EPT
_NEXTRA = (_E - _EXTRA_OFF) // _CHUNK
_ROWS_PT = 632
_NPAD = _N


def _matmul_body(x_ref, w_ref, o_ref):
    o_ref[...] = jnp.dot(x_ref[...], w_ref[...],
                         preferred_element_type=jnp.float32)


def _support_matmul(x, W):
    bm = 5000
    return pl.pallas_call(
        _matmul_body,
        out_shape=jax.ShapeDtypeStruct((_N, _D), jnp.float32),
        grid=(_N // bm,),
        in_specs=[pl.BlockSpec((bm, _D), lambda i: (i, 0)),
                  pl.BlockSpec((_D, _D), lambda i: (0, 0))],
        out_specs=pl.BlockSpec((bm, _D), lambda i: (i, 0)),
    )(x, W)


def _combine_body(p_ref, b_ref, o_ref):
    o_ref[...] = p_ref[0] + p_ref[1] + b_ref[...]


def _combine(partials, b2d):
    bm = 5000
    return pl.pallas_call(
        _combine_body,
        out_shape=jax.ShapeDtypeStruct((_N, _D), jnp.float32),
        grid=(_N // bm,),
        in_specs=[pl.BlockSpec((2, bm, _D), lambda i: (0, i, 0)),
                  pl.BlockSpec((1, _D), lambda i: (0, 0))],
        out_specs=pl.BlockSpec((bm, _D), lambda i: (i, 0)),
    )(partials, b2d)


def _sc_aggregate(support, ei, w):
    mesh = plsc.VectorSubcoreMesh(core_axis_name="c", subcore_axis_name="s")

    @functools.partial(
        pl.kernel,
        out_type=jax.ShapeDtypeStruct((2, _NPAD, _D), jnp.float32),
        mesh=mesh,
        scratch_types=[
            pltpu.VMEM((2, _CHUNK), jnp.int32),
            pltpu.VMEM((2, _CHUNK), jnp.int32),
            pltpu.VMEM((2, _CHUNK), jnp.int32),
            pltpu.VMEM((_CHUNK,), jnp.float32),
            pltpu.VMEM((_CHUNK,), jnp.float32),
            pltpu.VMEM((_CHUNK,), jnp.float32),
            pltpu.VMEM((_CHUNK, _D), jnp.float32),
            pltpu.VMEM((_CHUNK, _D), jnp.float32),
            pltpu.VMEM((_CHUNK, _D), jnp.float32),
            pltpu.VMEM_SHARED((_NPAD, _D), jnp.float32),
            pltpu.SemaphoreType.DMA,
            pltpu.SemaphoreType.DMA,
            pltpu.SemaphoreType.DMA,
            pltpu.SemaphoreType.DMA,
            pltpu.SemaphoreType.DMA,
            pltpu.SemaphoreType.DMA,
            pltpu.SemaphoreType.DMA,
            pltpu.SemaphoreType.DMA,
            pltpu.SemaphoreType.DMA,
        ],
    )
    def k(sup_hbm, ei_hbm, w_hbm, out_hbm,
          ebuf0, ebuf1, ebuf2, wbuf0, wbuf1, wbuf2, buf0, buf1, buf2, acc,
          esem0, esem1, esem2, gsem0, gsem1, gsem2, ssem0, ssem1, ssem2):
        cid = lax.axis_index("c")
        sid = lax.axis_index("s")
        wid = cid * 16 + sid
        base = wid * _EPT
        ebufs = (ebuf0, ebuf1, ebuf2)
        wbufs = (wbuf0, wbuf1, wbuf2)
        bufs = (buf0, buf1, buf2)
        esems = (esem0, esem1, esem2)
        gsems = (gsem0, gsem1, gsem2)
        ssems = (ssem0, ssem1, ssem2)

        def edge_start_at(off, q):
            pltpu.make_async_copy(ei_hbm.at[:, pl.ds(off, _CHUNK)],
                                  ebufs[q], esems[q]).start()
            pltpu.make_async_copy(w_hbm.at[pl.ds(off, _CHUNK)],
                                  wbufs[q], esems[q]).start()

        def edge_start(ci, q):
            edge_start_at(base + ci * _CHUNK, q)

        def edge_wait(q):
            pltpu.make_async_copy(ei_hbm.at[:, pl.ds(0, _CHUNK)],
                                  ebufs[q], esems[q]).wait()
            pltpu.make_async_copy(w_hbm.at[pl.ds(0, _CHUNK)],
                                  wbufs[q], esems[q]).wait()

        def gather_start(q):
            pltpu.make_async_copy(sup_hbm.at[ebufs[q].at[1]],
                                  bufs[q], gsems[q]).start()

        def gather_wait(q):
            pltpu.make_async_copy(sup_hbm.at[ebufs[q].at[1]],
                                  bufs[q], gsems[q]).wait()

        def scatter_start(q):
            pltpu.async_copy(bufs[q], acc.at[ebufs[q].at[0]], ssems[q],
                             add=True)

        def scatter_wait(q):
            pltpu.make_async_copy(sup_hbm.at[pl.ds(0, _CHUNK)], bufs[q],
                                  ssems[q]).wait()

        edge_start(0, 0)
        edge_start(1, 1)
        edge_wait(0)
        gather_start(0)
        edge_wait(1)
        gather_start(1)

        zeros16 = jnp.zeros((16,), jnp.float32)

        def zrow(r, carry):
            for j in range(8):
                buf2[r, pl.ds(j * 16, 16)] = zeros16
            return carry

        lax.fori_loop(0, _CHUNK, zrow, 0)
        row0 = jnp.minimum(sid * _ROWS_PT, _NPAD - _ROWS_PT)
        nz = _ROWS_PT // _CHUNK
        rem = _ROWS_PT - nz * _CHUNK
        for q in range(nz):
            pltpu.make_async_copy(
                buf2, acc.at[pl.ds(row0 + q * _CHUNK, _CHUNK)], gsem2).start()
        pltpu.make_async_copy(
            buf2.at[pl.ds(0, rem)],
            acc.at[pl.ds(row0 + nz * _CHUNK, rem)], gsem2).start()
        for q in range(nz):
            pltpu.make_async_copy(
                buf2, acc.at[pl.ds(row0, _CHUNK)], gsem2).wait()
        pltpu.make_async_copy(
            buf2.at[pl.ds(0, rem)], acc.at[pl.ds(row0, rem)], gsem2).wait()
        plsc.subcore_barrier()

        def mul_chunk(q):
            wbuf, buf = wbufs[q], bufs[q]

            @plsc.parallel_loop(0, _CHUNK // 16, unroll=2)
            def gbody(g):
                wv = wbuf[pl.ds(g * 16, 16)]
                for l in range(16):
                    wl = wv[l]
                    e = g * 16 + l
                    for j in range(8):
                        sl = pl.ds(j * 16, 16)
                        buf[e, sl] = buf[e, sl] * wl

        def phase(i, q, r):
            gather_wait(q)
            mul_chunk(q)
            scatter_start(q)

            @pl.when(i >= 1)
            def _ret():
                scatter_wait(r)

            @pl.when(i + 2 < _NCH)
            def _pre():
                edge_start(i + 2, r)
                edge_wait(r)
                gather_start(r)

        def body(kk, carry):
            i0 = kk * 3
            phase(i0, 0, 2)
            phase(i0 + 1, 1, 0)
            phase(i0 + 2, 2, 1)
            return carry

        lax.fori_loop(0, _NCH // 3, body, 0)
        scatter_wait((_NCH - 1) % 3)

        eidx = sid * 2 + cid
        @pl.when(sid < _NEXTRA // 2)
        def _extra():
            edge_start_at(_EXTRA_OFF + eidx * _CHUNK, 0)
            edge_wait(0)
            gather_start(0)
            gather_wait(0)
            mul_chunk(0)
            pltpu.sync_copy(buf0, acc.at[ebuf0.at[0]], add=True)

        plsc.subcore_barrier()
        pltpu.sync_copy(acc.at[pl.ds(row0, _ROWS_PT)],
                        out_hbm.at[cid, pl.ds(row0, _ROWS_PT)])

    return k(support, ei, w)


def kernel(x, edge_index, edge_weight, W, b):
    ei = jnp.asarray(edge_index, jnp.int32)
    w = jnp.asarray(edge_weight, jnp.float32)

    support = _support_matmul(x, W)
    partials = _sc_aggregate(support, ei, w)
    return _combine(partials, b.reshape(1, _D))
